# baseline (speedup 1.0000x reference)
import functools

import jax
import jax.numpy as jnp
from jax import lax
from jax.experimental import pallas as pl
from jax.experimental.pallas import tpu as pltpu
from jax.experimental.pallas import tpu_sc as plsc

_NEG_SLOPE = 0.2
_H = 4
_D = 32

_NC = 2
_NS = 16
_LANES = 16
_NW = _NC * _NS

_ROW = 128
_ROWS_PER_W = 80
_BATCH = 8
_NBATCH = _ROWS_PER_W // _BATCH
_N_ROWS = _NW * _ROWS_PER_W
_BE = _BATCH * _ROW
_BW = _BE * _H


def _tc_el_er(feat, w, al, ar):
    n = feat.shape[0]
    k = w.shape[1]

    def body(feat_ref, w_ref, al_ref, ar_ref, el_ref, er_ref):
        ft = jnp.dot(feat_ref[:], w_ref[:], preferred_element_type=jnp.float32)
        ii = lax.broadcasted_iota(jnp.int32, (k, _H), 0)
        jj = lax.broadcasted_iota(jnp.int32, (k, _H), 1)
        g = ((ii // _D) == jj).astype(jnp.float32)
        el_ref[:] = jnp.dot(ft * al_ref[:], g, preferred_element_type=jnp.float32)
        er_ref[:] = jnp.dot(ft * ar_ref[:], g, preferred_element_type=jnp.float32)

    return pl.pallas_call(
        body,
        out_shape=(
            jax.ShapeDtypeStruct((n, _H), jnp.float32),
            jax.ShapeDtypeStruct((n, _H), jnp.float32),
        ),
    )(feat, w, al, ar)


def _sc_attn(el_flat, er_flat, src_flat, dst_flat, zinit, e_total):
    nwords = el_flat.shape[0]
    n_padw = zinit.shape[0]
    zsl = n_padw // _NS
    epw = _ROWS_PER_W * _ROW
    wpw = epw * _H
    ewords = _N_ROWS * _ROW * _H
    mesh = plsc.VectorSubcoreMesh(
        core_axis_name="c", subcore_axis_name="s",
        num_cores=_NC, num_subcores=_NS)

    @functools.partial(
        pl.kernel,
        out_type=(
            jax.ShapeDtypeStruct((ewords,), jnp.float32),
            jax.ShapeDtypeStruct((ewords,), jnp.float32),
        ),
        mesh=mesh,
        compiler_params=pltpu.CompilerParams(needs_layout_passes=False),
        scratch_types=[
            pltpu.VMEM((nwords,), jnp.float32),
            pltpu.VMEM((nwords,), jnp.float32),
            pltpu.VMEM((2 * _BE,), jnp.int32),
            pltpu.VMEM((2 * _BE,), jnp.int32),
            pltpu.VMEM((2 * _BW,), jnp.float32),
            pltpu.VMEM((2, _BATCH * _H, _ROW), jnp.int32),
            pltpu.VMEM((2 * _BW,), jnp.float32),
            pltpu.VMEM_SHARED((n_padw,), jnp.float32),
            pltpu.SemaphoreType.DMA,
            pltpu.SemaphoreType.DMA,
            pltpu.SemaphoreType.DMA,
        ],
    )
    def k(el_hbm, er_hbm, src_hbm, dst_hbm, z_hbm,
          s_hbm, a_hbm,
          el_tab, er_tab, srcb, dstb, s_buf, idx_buf, a_buf, denom_sh,
          sem_sc, sem_w, sem_r):
        cid = lax.axis_index("c")
        sid = lax.axis_index("s")
        wid = cid * _NS + sid
        mid = (1 - cid) * _NS + sid
        e_own = wid * epw
        e_mir = mid * epw
        w_own = wid * wpw

        pltpu.sync_copy(el_hbm, el_tab)
        pltpu.sync_copy(er_hbm, er_tab)
        pltpu.sync_copy(z_hbm.at[pl.ds(sid * zsl, zsl)],
                        denom_sh.at[pl.ds(sid * zsl, zsl)])
        plsc.subcore_barrier()

        lane = lax.iota(jnp.int32, _LANES)
        rep = lane >> 2
        hh = lane & 3

        def fire_reads(ebase, b, d):
            pltpu.async_copy(src_hbm.at[pl.ds(ebase + b * _BE, _BE)],
                             srcb.at[pl.ds(d * _BE, _BE)], sem_r)
            pltpu.async_copy(dst_hbm.at[pl.ds(ebase + b * _BE, _BE)],
                             dstb.at[pl.ds(d * _BE, _BE)], sem_r)

        def wait_reads(ebase, b, d):
            pltpu.make_async_copy(src_hbm.at[pl.ds(ebase + b * _BE, _BE)],
                                  srcb.at[pl.ds(d * _BE, _BE)], sem_r).wait()
            pltpu.make_async_copy(dst_hbm.at[pl.ds(ebase + b * _BE, _BE)],
                                  dstb.at[pl.ds(d * _BE, _BE)], sem_r).wait()

        def drain_scatter(d):
            def waitfn(t, carry2):
                pltpu.make_async_copy(
                    s_buf.at[pl.ds(d * _BW + t * _ROW, _ROW)],
                    denom_sh.at[idx_buf.at[d, t]],
                    sem_sc).wait()
                return carry2

            lax.fori_loop(0, _BATCH * _H, waitfn, 0)

        def compute_batch(ebase, b, d, write_s):
            def rowfn(q, carry2):
                dbase = jnp.broadcast_to(d * _BE + q * _ROW, (_LANES,))
                for g in range(_ROW // 4):
                    eidx = g * 4 + rep
                    srcw = plsc.load_gather(srcb, [dbase + eidx])
                    dstw = plsc.load_gather(dstb, [dbase + eidx])
                    elw = plsc.load_gather(el_tab, [(srcw << 2) + hh])
                    erw = plsc.load_gather(er_tab, [(dstw << 2) + hh])
                    e = elw + erw
                    e = jnp.where(e >= 0.0, e, e * _NEG_SLOPE)
                    sw = jnp.exp(e)
                    gid = ebase + (b * _BATCH + q) * _ROW + eidx
                    sw = jnp.where(gid < e_total, sw, 0.0)
                    t = q * _H + g // 8
                    c = (g % 8) * _LANES
                    s_buf[pl.ds(d * _BW + t * _ROW + c, _LANES)] = sw
                    idx_buf[d, t, pl.ds(c, _LANES)] = (dstw << 2) + hh
                for u in range(_H):
                    tt = q * _H + u
                    pltpu.async_copy(
                        s_buf.at[pl.ds(d * _BW + tt * _ROW, _ROW)],
                        denom_sh.at[idx_buf.at[d, tt]],
                        sem_sc, add=True)
                return carry2

            lax.fori_loop(0, _BATCH, rowfn, 0)
            if write_s:
                pltpu.async_copy(
                    s_buf.at[pl.ds(d * _BW, _BW)],
                    s_hbm.at[pl.ds(w_own + b * _BW, _BW)], sem_w)

        def drain_write(b):
            pltpu.make_async_copy(
                s_buf.at[pl.ds((b & 1) * _BW, _BW)],
                s_hbm.at[pl.ds(w_own + b * _BW, _BW)], sem_w).wait()

        fire_reads(e_own, 0, 0)

        def batch_a1(b, carry):
            d = b & 1

            @pl.when(b + 1 < _NBATCH)
            def _():
                fire_reads(e_own, b + 1, 1 - d)

            wait_reads(e_own, b, d)

            @pl.when(b >= 2)
            def _():
                drain_scatter(d)
                drain_write(b - 2)

            compute_batch(e_own, b, d, True)
            return carry

        lax.fori_loop(0, _NBATCH, batch_a1, 0)

        fire_reads(e_mir, 0, 0)

        def batch_a2(b, carry):
            d = b & 1

            @pl.when(b + 1 < _NBATCH)
            def _():
                fire_reads(e_mir, b + 1, 1 - d)

            wait_reads(e_mir, b, d)
            drain_scatter(d)

            @pl.when(b < 2)
            def _():
                drain_write(_NBATCH - 2 + b)

            compute_batch(e_mir, b, d, False)
            return carry

        lax.fori_loop(0, _NBATCH, batch_a2, 0)
        drain_scatter(0)
        drain_scatter(1)
        plsc.subcore_barrier()

        def recip(base, nw):
            pltpu.sync_copy(denom_sh.at[pl.ds(base, nw)],
                            s_buf.at[pl.ds(0, nw)])
            for j in range(nw // _LANES):
                v = s_buf[pl.ds(j * _LANES, _LANES)]
                er_tab[pl.ds(base + j * _LANES, _LANES)] = 1.0 / v

        nfull = nwords // _BW
        tailw = nwords - nfull * _BW

        def chunk(ci, carry):
            recip(ci * _BW, _BW)
            return carry

        lax.fori_loop(0, nfull, chunk, 0)
        if tailw:
            recip(nfull * _BW, tailw)

        def fire_b_reads(b, d):
            pltpu.async_copy(s_hbm.at[pl.ds(w_own + b * _BW, _BW)],
                             s_buf.at[pl.ds(d * _BW, _BW)], sem_r)
            pltpu.async_copy(dst_hbm.at[pl.ds(e_own + b * _BE, _BE)],
                             dstb.at[pl.ds(d * _BE, _BE)], sem_r)

        def wait_b_reads(b, d):
            pltpu.make_async_copy(s_hbm.at[pl.ds(w_own + b * _BW, _BW)],
                                  s_buf.at[pl.ds(d * _BW, _BW)], sem_r).wait()
            pltpu.make_async_copy(dst_hbm.at[pl.ds(e_own + b * _BE, _BE)],
                                  dstb.at[pl.ds(d * _BE, _BE)], sem_r).wait()

        def drain_a(b):
            pltpu.make_async_copy(
                a_buf.at[pl.ds((b & 1) * _BW, _BW)],
                a_hbm.at[pl.ds(w_own + b * _BW, _BW)], sem_w).wait()

        fire_b_reads(0, 0)

        def batch_b(b, carry):
            d = b & 1

            @pl.when(b + 1 < _NBATCH)
            def _():
                fire_b_reads(b + 1, 1 - d)

            wait_b_reads(b, d)

            @pl.when(b >= 2)
            def _():
                drain_a(b - 2)

            def rowfn(q, carry2):
                dbase = jnp.broadcast_to(d * _BE + q * _ROW, (_LANES,))
                for g in range(_ROW // 4):
                    eidx = g * 4 + rep
                    dstw = plsc.load_gather(dstb, [dbase + eidx])
                    dv = plsc.load_gather(er_tab, [(dstw << 2) + hh])
                    t = q * _H + g // 8
                    c = (g % 8) * _LANES
                    woff = d * _BW + t * _ROW + c
                    sv = s_buf[pl.ds(woff, _LANES)]
                    a_buf[pl.ds(woff, _LANES)] = sv * dv
                return carry2

            lax.fori_loop(0, _BATCH, rowfn, 0)
            pltpu.async_copy(
                a_buf.at[pl.ds(d * _BW, _BW)],
                a_hbm.at[pl.ds(w_own + b * _BW, _BW)], sem_w)
            return carry

        lax.fori_loop(0, _NBATCH, batch_b, 0)
        drain_a(_NBATCH - 2)
        drain_a(_NBATCH - 1)

    return k(el_flat, er_flat, src_flat, dst_flat, zinit)


def kernel(feat, edge_index, W, attn_l, attn_r):
    n = feat.shape[0]
    e = edge_index.shape[1]

    al = attn_l.reshape(1, _H * _D)
    ar = attn_r.reshape(1, _H * _D)
    el, er = _tc_el_er(feat, W, al, ar)

    src = edge_index[0].astype(jnp.int32)
    dst = edge_index[1].astype(jnp.int32)
    e_pad = _N_ROWS * _ROW
    pad = e_pad - e
    zpad = jnp.zeros((pad,), jnp.int32)
    src_flat = jnp.concatenate([src, zpad])
    dst_flat = jnp.concatenate([dst, zpad])

    n_padw = ((n * _H + 4095) // 4096) * 4096
    z = jnp.zeros((n_padw,), jnp.float32)

    _, a_flat = _sc_attn(el.reshape(-1), er.reshape(-1),
                         src_flat, dst_flat, z, e)
    return a_flat.reshape(e_pad, _H)[:e].reshape(e, _H, 1)
